# baseline (speedup 1.0000x reference)
import jax
import jax.numpy as jnp
from jax import lax
from jax.experimental import pallas as pl
from jax.experimental.pallas import tpu as pltpu
from jax.experimental.pallas import tpu_sc as plsc

_BATCH = 64
_NUM_POS = 196
_HIDDEN = 768
_P_OUT = _NUM_POS + 1

_B_SC = 16
_B_TC = _BATCH - _B_SC

_LANES = 16
_ROWS = 7
_MAIN = 28
_NB = 8
_NGROUPS = _B_SC // _NB
_VECS = _HIDDEN // _LANES


def _sc_body(feats_hbm, qw_hbm, pw_hbm, out_hbm,
             buf0, buf1, pwin, pw5, qw_buf, row0_buf,
             si0, si1, so0, so1):
    bufs = [buf0, buf1]
    sems_in = [si0, si1]
    sems_out = [so0, so1]
    c = lax.axis_index("c")
    s = lax.axis_index("s")
    wid = s * 2 + c

    def in_dma(g, p0):
        return pltpu.async_copy(
            feats_hbm.at[pl.ds(p0, _ROWS), pl.ds(_B_TC + g * _NB, _NB), :],
            bufs[g % 2], sems_in[g % 2])

    def out_dma(g, p0):
        return pltpu.async_copy(
            bufs[g % 2],
            out_hbm.at[pl.ds(p0 + 1, _ROWS), pl.ds(g * _NB, _NB), :],
            sems_out[g % 2])

    def run_groups(p0, compute):
        h_out = [None] * _NGROUPS
        h_in = [in_dma(g, p0) for g in range(_NGROUPS)]
        for g in range(_NGROUPS):
            h_in[g].wait()
            compute(g)
            h_out[g] = out_dma(g, p0)
        for g in range(_NGROUPS):
            h_out[g].wait()

    @pl.when(wid < _MAIN - 1)
    def _main():
        p0 = wid * _ROWS
        a0 = ((p0 + 1) // 8) * 8
        widx = p0 + 1 - a0
        pltpu.sync_copy(pw_hbm.at[pl.ds(a0, 16), :], pwin)

        def compute(g):
            buf = bufs[g % 2]

            def vec_step(v, _):
                co = v * _LANES
                for r in range(_ROWS):
                    pwv = pwin[widx + r, pl.ds(co, _LANES)]
                    for b in range(_NB):
                        buf[r, b, pl.ds(co, _LANES)] = (
                            buf[r, b, pl.ds(co, _LANES)] + pwv)
                return 0

            lax.fori_loop(0, _VECS, vec_step, 0)

        run_groups(p0, compute)

    @pl.when(wid == _MAIN - 1)
    def _last():
        p0 = (_MAIN - 1) * _ROWS
        pltpu.sync_copy(pw_hbm.at[pl.ds(184, 8), :], pwin.at[pl.ds(0, 8), :])
        pltpu.sync_copy(pw_hbm.at[pl.ds(192, 5), :], pw5)

        def compute(g):
            buf = bufs[g % 2]

            def vec_step(v, _):
                co = v * _LANES
                for r in range(_ROWS):
                    if r < 2:
                        pwv = pwin[6 + r, pl.ds(co, _LANES)]
                    else:
                        pwv = pw5[r - 2, pl.ds(co, _LANES)]
                    for b in range(_NB):
                        buf[r, b, pl.ds(co, _LANES)] = (
                            buf[r, b, pl.ds(co, _LANES)] + pwv)
                return 0

            lax.fori_loop(0, _VECS, vec_step, 0)

        run_groups(p0, compute)

    @pl.when(wid == _MAIN)
    def _row0():
        pltpu.sync_copy(pw_hbm.at[pl.ds(0, 8), :], pwin.at[pl.ds(0, 8), :])
        pltpu.sync_copy(qw_hbm, qw_buf)

        def vec_step(v, _):
            co = v * _LANES
            val = qw_buf[0, pl.ds(co, _LANES)] + pwin[0, pl.ds(co, _LANES)]
            for b in range(_NB):
                row0_buf[0, b, pl.ds(co, _LANES)] = val
            return 0

        lax.fori_loop(0, _VECS, vec_step, 0)
        hs = [pltpu.async_copy(
                  row0_buf, out_hbm.at[pl.ds(0, 1), pl.ds(g * _NB, _NB), :],
                  sems_out[g % 2])
              for g in range(_NGROUPS)]
        for h in hs:
            h.wait()


def _sc_call(feats_t, quality_weight, position_weight):
    mesh = plsc.VectorSubcoreMesh(core_axis_name="c", subcore_axis_name="s")
    run = pl.kernel(
        _sc_body,
        out_type=jax.ShapeDtypeStruct((_P_OUT, _B_SC, _HIDDEN), jnp.float32),
        mesh=mesh,
        scratch_types=[
            pltpu.VMEM((_ROWS, _NB, _HIDDEN), jnp.float32),
            pltpu.VMEM((_ROWS, _NB, _HIDDEN), jnp.float32),
            pltpu.VMEM((16, _HIDDEN), jnp.float32),
            pltpu.VMEM((5, _HIDDEN), jnp.float32),
            pltpu.VMEM((1, _HIDDEN), jnp.float32),
            pltpu.VMEM((1, _NB, _HIDDEN), jnp.float32),
            pltpu.SemaphoreType.DMA,
            pltpu.SemaphoreType.DMA,
            pltpu.SemaphoreType.DMA,
            pltpu.SemaphoreType.DMA,
        ],
        compiler_params=pltpu.CompilerParams(use_tc_tiling_on_sc=True),
    )
    return run(feats_t, quality_weight, position_weight)


def _tc_body(feats_ref, qw_ref, pw_ref, out_ref):
    p = pl.program_id(0)

    @pl.when(p == 0)
    def _():
        out_ref[...] = jnp.broadcast_to(
            (qw_ref[...] + pw_ref[pl.ds(0, 1), :])[:, None, :],
            (1, _B_TC, _HIDDEN))

    @pl.when(p > 0)
    def _():
        out_ref[...] = feats_ref[...] + pw_ref[pl.ds(p, 1), :][:, None, :]


def _tc_call(feats_t, quality_weight, position_weight):
    return pl.pallas_call(
        _tc_body,
        grid=(_P_OUT,),
        in_specs=[
            pl.BlockSpec((1, _B_TC, _HIDDEN),
                         lambda p: (jnp.maximum(p - 1, 0), 0, 0)),
            pl.BlockSpec((1, _HIDDEN), lambda p: (0, 0)),
            pl.BlockSpec((_P_OUT, _HIDDEN), lambda p: (0, 0)),
        ],
        out_specs=pl.BlockSpec((1, _B_TC, _HIDDEN), lambda p: (p, 0, 0)),
        out_shape=jax.ShapeDtypeStruct((_P_OUT, _BATCH, _HIDDEN), jnp.float32),
        compiler_params=pltpu.CompilerParams(
            dimension_semantics=("arbitrary",)),
    )(feats_t, quality_weight, position_weight)


@jax.jit
def kernel(feats, quality_weight, position_weight):
    feats_t = jnp.transpose(feats, (1, 0, 2))
    sc_out = _sc_call(feats_t, quality_weight, position_weight)
    tc_out = _tc_call(feats_t, quality_weight, position_weight)
    out_t = lax.dynamic_update_slice(tc_out, sc_out, (0, _B_TC, 0))
    return jnp.transpose(out_t, (1, 0, 2))


# speedup vs baseline: 2.5846x; 2.5846x over previous
import jax
import jax.numpy as jnp
from jax import lax
from jax.experimental import pallas as pl
from jax.experimental.pallas import tpu as pltpu
from jax.experimental.pallas import tpu_sc as plsc

_BATCH = 64
_NUM_POS = 196
_HIDDEN = 768
_P_OUT = _NUM_POS + 1

_B_SC = 16
_B_TC = _BATCH - _B_SC

_LANES = 16
_ROWS = 7
_MAIN = 28
_NB = 8
_NGROUPS = _B_SC // _NB
_VECS = _HIDDEN // _LANES


def _sc_body(feats_hbm, qw_hbm, pw_hbm, out_hbm,
             buf0, buf1, pwin, pw5, qw_buf, row0_buf,
             si0, si1, so0, so1):
    bufs = [buf0, buf1]
    sems_in = [si0, si1]
    sems_out = [so0, so1]
    c = lax.axis_index("c")
    s = lax.axis_index("s")
    wid = s * 2 + c

    def in_dma(g, p0):
        return pltpu.async_copy(
            feats_hbm.at[pl.ds(p0, _ROWS), pl.ds(_B_TC + g * _NB, _NB), :],
            bufs[g % 2], sems_in[g % 2])

    def out_dma(g, p0):
        return pltpu.async_copy(
            bufs[g % 2],
            out_hbm.at[pl.ds(p0 + 1, _ROWS), pl.ds(g * _NB, _NB), :],
            sems_out[g % 2])

    def run_groups(p0, compute):
        h_out = [None] * _NGROUPS
        h_in = [in_dma(g, p0) for g in range(_NGROUPS)]
        for g in range(_NGROUPS):
            h_in[g].wait()
            compute(g)
            h_out[g] = out_dma(g, p0)
        for g in range(_NGROUPS):
            h_out[g].wait()

    @pl.when(wid < _MAIN - 1)
    def _main():
        p0 = wid * _ROWS
        a0 = ((p0 + 1) // 8) * 8
        widx = p0 + 1 - a0
        pltpu.sync_copy(pw_hbm.at[pl.ds(a0, 16), :], pwin)

        def compute(g):
            buf = bufs[g % 2]

            def vec_step(v, _):
                co = v * _LANES
                for r in range(_ROWS):
                    pwv = pwin[widx + r, pl.ds(co, _LANES)]
                    for b in range(_NB):
                        buf[r, b, pl.ds(co, _LANES)] = (
                            buf[r, b, pl.ds(co, _LANES)] + pwv)
                return 0

            lax.fori_loop(0, _VECS, vec_step, 0)

        run_groups(p0, compute)

    @pl.when(wid == _MAIN - 1)
    def _last():
        p0 = (_MAIN - 1) * _ROWS
        pltpu.sync_copy(pw_hbm.at[pl.ds(184, 8), :], pwin.at[pl.ds(0, 8), :])
        pltpu.sync_copy(pw_hbm.at[pl.ds(192, 5), :], pw5)

        def compute(g):
            buf = bufs[g % 2]

            def vec_step(v, _):
                co = v * _LANES
                for r in range(_ROWS):
                    if r < 2:
                        pwv = pwin[6 + r, pl.ds(co, _LANES)]
                    else:
                        pwv = pw5[r - 2, pl.ds(co, _LANES)]
                    for b in range(_NB):
                        buf[r, b, pl.ds(co, _LANES)] = (
                            buf[r, b, pl.ds(co, _LANES)] + pwv)
                return 0

            lax.fori_loop(0, _VECS, vec_step, 0)

        run_groups(p0, compute)

    @pl.when(wid == _MAIN)
    def _row0():
        pltpu.sync_copy(pw_hbm.at[pl.ds(0, 8), :], pwin.at[pl.ds(0, 8), :])
        pltpu.sync_copy(qw_hbm, qw_buf)

        def vec_step(v, _):
            co = v * _LANES
            val = qw_buf[0, pl.ds(co, _LANES)] + pwin[0, pl.ds(co, _LANES)]
            for b in range(_NB):
                row0_buf[0, b, pl.ds(co, _LANES)] = val
            return 0

        lax.fori_loop(0, _VECS, vec_step, 0)
        hs = [pltpu.async_copy(
                  row0_buf, out_hbm.at[pl.ds(0, 1), pl.ds(g * _NB, _NB), :],
                  sems_out[g % 2])
              for g in range(_NGROUPS)]
        for h in hs:
            h.wait()


def _sc_call(feats_t, quality_weight, position_weight):
    mesh = plsc.VectorSubcoreMesh(core_axis_name="c", subcore_axis_name="s")
    run = pl.kernel(
        _sc_body,
        out_type=jax.ShapeDtypeStruct((_P_OUT, _B_SC, _HIDDEN), jnp.float32),
        mesh=mesh,
        scratch_types=[
            pltpu.VMEM((_ROWS, _NB, _HIDDEN), jnp.float32),
            pltpu.VMEM((_ROWS, _NB, _HIDDEN), jnp.float32),
            pltpu.VMEM((16, _HIDDEN), jnp.float32),
            pltpu.VMEM((5, _HIDDEN), jnp.float32),
            pltpu.VMEM((1, _HIDDEN), jnp.float32),
            pltpu.VMEM((1, _NB, _HIDDEN), jnp.float32),
            pltpu.SemaphoreType.DMA,
            pltpu.SemaphoreType.DMA,
            pltpu.SemaphoreType.DMA,
            pltpu.SemaphoreType.DMA,
        ],
        compiler_params=pltpu.CompilerParams(use_tc_tiling_on_sc=True),
    )
    return run(feats_t, quality_weight, position_weight)


def _tc_body(feats_ref, qw_ref, pw_ref, out_ref):
    out_ref[pl.ds(1, _NUM_POS), :, :] = (
        feats_ref[...] + pw_ref[pl.ds(1, _NUM_POS), :][:, None, :])
    out_ref[pl.ds(0, 1), :, :] = jnp.broadcast_to(
        (qw_ref[...] + pw_ref[pl.ds(0, 1), :])[:, None, :],
        (1, _TCB, _HIDDEN))


_TCB = 8


def _tc_call(feats_t, quality_weight, position_weight):
    return pl.pallas_call(
        _tc_body,
        grid=(_B_TC // _TCB,),
        in_specs=[
            pl.BlockSpec((_NUM_POS, _TCB, _HIDDEN), lambda g: (0, g, 0)),
            pl.BlockSpec((1, _HIDDEN), lambda g: (0, 0)),
            pl.BlockSpec((_P_OUT, _HIDDEN), lambda g: (0, 0)),
        ],
        out_specs=pl.BlockSpec((_P_OUT, _TCB, _HIDDEN), lambda g: (0, g, 0)),
        out_shape=jax.ShapeDtypeStruct((_P_OUT, _BATCH, _HIDDEN), jnp.float32),
        compiler_params=pltpu.CompilerParams(
            dimension_semantics=("arbitrary",)),
    )(feats_t, quality_weight, position_weight)


@jax.jit
def kernel(feats, quality_weight, position_weight):
    feats_t = jnp.transpose(feats, (1, 0, 2))
    sc_out = _sc_call(feats_t, quality_weight, position_weight)
    tc_out = _tc_call(feats_t, quality_weight, position_weight)
    out_t = lax.dynamic_update_slice(tc_out, sc_out, (0, _B_TC, 0))
    return jnp.transpose(out_t, (1, 0, 2))


# speedup vs baseline: 2.8744x; 1.1121x over previous
import jax
import jax.numpy as jnp
from jax import lax
from jax.experimental import pallas as pl
from jax.experimental.pallas import tpu as pltpu
from jax.experimental.pallas import tpu_sc as plsc

_BATCH = 64
_NUM_POS = 196
_HIDDEN = 768
_P_OUT = _NUM_POS + 1

_B_SC = 8
_B_TC = _BATCH - _B_SC

_LANES = 16
_ROWS = 7
_MAIN = 28
_NB = 8
_NGROUPS = _B_SC // _NB
_VECS = _HIDDEN // _LANES


def _sc_body(feats_hbm, qw_hbm, pw_hbm, out_hbm,
             buf0, buf1, pwin, pw5, qw_buf, row0_buf,
             si0, si1, so0, so1):
    bufs = [buf0, buf1]
    sems_in = [si0, si1]
    sems_out = [so0, so1]
    c = lax.axis_index("c")
    s = lax.axis_index("s")
    wid = s * 2 + c

    def in_dma(g, p0):
        return pltpu.async_copy(
            feats_hbm.at[pl.ds(p0, _ROWS), pl.ds(_B_TC + g * _NB, _NB), :],
            bufs[g % 2], sems_in[g % 2])

    def out_dma(g, p0):
        return pltpu.async_copy(
            bufs[g % 2],
            out_hbm.at[pl.ds(p0 + 1, _ROWS), pl.ds(g * _NB, _NB), :],
            sems_out[g % 2])

    def run_groups(p0, compute):
        h_out = [None] * _NGROUPS
        h_in = [in_dma(g, p0) for g in range(_NGROUPS)]
        for g in range(_NGROUPS):
            h_in[g].wait()
            compute(g)
            h_out[g] = out_dma(g, p0)
        for g in range(_NGROUPS):
            h_out[g].wait()

    @pl.when(wid < _MAIN - 1)
    def _main():
        p0 = wid * _ROWS
        a0 = ((p0 + 1) // 8) * 8
        widx = p0 + 1 - a0
        pltpu.sync_copy(pw_hbm.at[pl.ds(a0, 16), :], pwin)

        def compute(g):
            buf = bufs[g % 2]

            def vec_step(v, _):
                co = v * _LANES
                for r in range(_ROWS):
                    pwv = pwin[widx + r, pl.ds(co, _LANES)]
                    for b in range(_NB):
                        buf[r, b, pl.ds(co, _LANES)] = (
                            buf[r, b, pl.ds(co, _LANES)] + pwv)
                return 0

            lax.fori_loop(0, _VECS, vec_step, 0)

        run_groups(p0, compute)

    @pl.when(wid == _MAIN - 1)
    def _last():
        p0 = (_MAIN - 1) * _ROWS
        pltpu.sync_copy(pw_hbm.at[pl.ds(184, 8), :], pwin.at[pl.ds(0, 8), :])
        pltpu.sync_copy(pw_hbm.at[pl.ds(192, 5), :], pw5)

        def compute(g):
            buf = bufs[g % 2]

            def vec_step(v, _):
                co = v * _LANES
                for r in range(_ROWS):
                    if r < 2:
                        pwv = pwin[6 + r, pl.ds(co, _LANES)]
                    else:
                        pwv = pw5[r - 2, pl.ds(co, _LANES)]
                    for b in range(_NB):
                        buf[r, b, pl.ds(co, _LANES)] = (
                            buf[r, b, pl.ds(co, _LANES)] + pwv)
                return 0

            lax.fori_loop(0, _VECS, vec_step, 0)

        run_groups(p0, compute)

    @pl.when(wid == _MAIN)
    def _row0():
        pltpu.sync_copy(pw_hbm.at[pl.ds(0, 8), :], pwin.at[pl.ds(0, 8), :])
        pltpu.sync_copy(qw_hbm, qw_buf)

        def vec_step(v, _):
            co = v * _LANES
            val = qw_buf[0, pl.ds(co, _LANES)] + pwin[0, pl.ds(co, _LANES)]
            for b in range(_NB):
                row0_buf[0, b, pl.ds(co, _LANES)] = val
            return 0

        lax.fori_loop(0, _VECS, vec_step, 0)
        hs = [pltpu.async_copy(
                  row0_buf, out_hbm.at[pl.ds(0, 1), pl.ds(g * _NB, _NB), :],
                  sems_out[g % 2])
              for g in range(_NGROUPS)]
        for h in hs:
            h.wait()


def _sc_call(feats_t, quality_weight, position_weight):
    mesh = plsc.VectorSubcoreMesh(core_axis_name="c", subcore_axis_name="s")
    run = pl.kernel(
        _sc_body,
        out_type=jax.ShapeDtypeStruct((_P_OUT, _B_SC, _HIDDEN), jnp.float32),
        mesh=mesh,
        scratch_types=[
            pltpu.VMEM((_ROWS, _NB, _HIDDEN), jnp.float32),
            pltpu.VMEM((_ROWS, _NB, _HIDDEN), jnp.float32),
            pltpu.VMEM((16, _HIDDEN), jnp.float32),
            pltpu.VMEM((5, _HIDDEN), jnp.float32),
            pltpu.VMEM((1, _HIDDEN), jnp.float32),
            pltpu.VMEM((1, _NB, _HIDDEN), jnp.float32),
            pltpu.SemaphoreType.DMA,
            pltpu.SemaphoreType.DMA,
            pltpu.SemaphoreType.DMA,
            pltpu.SemaphoreType.DMA,
        ],
        compiler_params=pltpu.CompilerParams(use_tc_tiling_on_sc=True),
    )
    return run(feats_t, quality_weight, position_weight)


def _tc_body(feats_ref, qw_ref, pw_ref, out_ref):
    out_ref[pl.ds(1, _NUM_POS), :, :] = (
        feats_ref[...] + pw_ref[pl.ds(1, _NUM_POS), :][:, None, :])
    out_ref[pl.ds(0, 1), :, :] = jnp.broadcast_to(
        (qw_ref[...] + pw_ref[pl.ds(0, 1), :])[:, None, :],
        (1, _TCB, _HIDDEN))


_TCB = 8


def _tc_call(feats_t, quality_weight, position_weight):
    return pl.pallas_call(
        _tc_body,
        grid=(_B_TC // _TCB,),
        in_specs=[
            pl.BlockSpec((_NUM_POS, _TCB, _HIDDEN), lambda g: (0, g, 0)),
            pl.BlockSpec((1, _HIDDEN), lambda g: (0, 0)),
            pl.BlockSpec((_P_OUT, _HIDDEN), lambda g: (0, 0)),
        ],
        out_specs=pl.BlockSpec((_P_OUT, _TCB, _HIDDEN), lambda g: (0, g, 0)),
        out_shape=jax.ShapeDtypeStruct((_P_OUT, _BATCH, _HIDDEN), jnp.float32),
        compiler_params=pltpu.CompilerParams(
            dimension_semantics=("arbitrary",)),
    )(feats_t, quality_weight, position_weight)


@jax.jit
def kernel(feats, quality_weight, position_weight):
    feats_t = jnp.transpose(feats, (1, 0, 2))
    sc_out = _sc_call(feats_t, quality_weight, position_weight)
    tc_out = _tc_call(feats_t, quality_weight, position_weight)
    out_t = lax.dynamic_update_slice(tc_out, sc_out, (0, _B_TC, 0))
    return jnp.transpose(out_t, (1, 0, 2))
